# baseline (speedup 1.0000x reference)
import jax
import jax.numpy as jnp
from jax import lax
from jax.experimental import pallas as pl
from jax.experimental.pallas import tpu as pltpu
from jax.experimental.pallas import tpu_sc as plsc

_N = 16777216
_NC = 2
_SHARD = _N // _NC
_CHUNK = 65536
_NCHUNK = _SHARD // _CHUNK
_NBUF = 28
_D = 14
_HEAD = 128


def _scalar_body(x_hbm, o_hbm):
    def inner(*refs):
        bufs = refs[:_NBUF]
        buf0 = refs[_NBUF]
        insems = refs[_NBUF + 1:2 * _NBUF + 1]
        outsems = refs[2 * _NBUF + 1:]
        wid = lax.axis_index("c")
        base = wid * _SHARD

        def in_cp(c):
            if c == 0:
                return pltpu.make_async_copy(
                    x_hbm.at[pl.ds(base + _HEAD, _CHUNK - _HEAD)],
                    buf0, insems[0])
            return pltpu.make_async_copy(
                x_hbm.at[pl.ds(base + c * _CHUNK, _CHUNK)],
                bufs[c % _NBUF], insems[c % _NBUF])

        def out_cp(c):
            if c == 0:
                return pltpu.make_async_copy(
                    buf0,
                    o_hbm.at[pl.ds(base + _HEAD, _CHUNK - _HEAD)], outsems[0])
            return pltpu.make_async_copy(
                bufs[c % _NBUF],
                o_hbm.at[pl.ds(base + c * _CHUNK, _CHUNK)], outsems[c % _NBUF])

        for c in range(_D):
            in_cp(c).start()
        for c in range(_NCHUNK):
            in_cp(c).wait()
            out_cp(c).start()
            nxt = c + _D
            if nxt < _NCHUNK:
                if nxt >= _NBUF:
                    out_cp(nxt - _NBUF).wait()
                in_cp(nxt).start()
        for c in range(_NCHUNK - _NBUF, _NCHUNK):
            out_cp(c).wait()

    pl.run_scoped(
        inner,
        *([pltpu.VMEM_SHARED((_CHUNK,), jnp.float32)] * _NBUF),
        pltpu.VMEM_SHARED((_CHUNK - _HEAD,), jnp.float32),
        *([pltpu.SemaphoreType.DMA] * (2 * _NBUF)),
    )


def _vector_body(x_hbm, o_hbm):
    def inner(buf16):
        core = lax.axis_index("c")
        sid = lax.axis_index("s")
        base = core * _SHARD

        @pl.when(sid == 0)
        def _head():
            pltpu.sync_copy(x_hbm.at[pl.ds(base, _HEAD)], buf16)

            @pl.when(core == 0)
            def _patch():
                i = lax.iota(jnp.int32, 16)
                v = buf16[pl.ds(0, 16)]
                buf16[pl.ds(0, 16)] = jnp.where(i == 0, 10.0,
                                      jnp.where(i == 1, 30.0,
                                      jnp.where(i == 2, 20.0,
                                      jnp.where(i == 3, 40.0, v))))

            pltpu.sync_copy(buf16, o_hbm.at[pl.ds(base, _HEAD)])

    pl.run_scoped(inner, pltpu.VMEM((_HEAD,), jnp.float32))


def kernel(data):
    s_mesh = plsc.ScalarSubcoreMesh(axis_name="c")
    v_mesh = plsc.VectorSubcoreMesh(core_axis_name="c", subcore_axis_name="s")
    f = pl.kernel(
        body=[_scalar_body, _vector_body],
        mesh=[s_mesh, v_mesh],
        out_type=jax.ShapeDtypeStruct((_N,), jnp.float32),
    )
    return f(data)


# speedup vs baseline: 1.0154x; 1.0154x over previous
import jax
import jax.numpy as jnp
from jax import lax
from jax.experimental import pallas as pl
from jax.experimental.pallas import tpu as pltpu
from jax.experimental.pallas import tpu_sc as plsc

_N = 16777216
_NC = 2
_SHARD = _N // _NC
_CHUNK = 131072
_NCHUNK = _SHARD // _CHUNK
_NBUF = 14
_D = 7
_HEAD = 128


def _scalar_body(x_hbm, o_hbm):
    def inner(*refs):
        bufs = refs[:_NBUF]
        buf0 = refs[_NBUF]
        insems = refs[_NBUF + 1:2 * _NBUF + 1]
        outsems = refs[2 * _NBUF + 1:]
        wid = lax.axis_index("c")
        base = wid * _SHARD

        def in_cp(c):
            if c == 0:
                return pltpu.make_async_copy(
                    x_hbm.at[pl.ds(base + _HEAD, _CHUNK - _HEAD)],
                    buf0, insems[0])
            return pltpu.make_async_copy(
                x_hbm.at[pl.ds(base + c * _CHUNK, _CHUNK)],
                bufs[c % _NBUF], insems[c % _NBUF])

        def out_cp(c):
            if c == 0:
                return pltpu.make_async_copy(
                    buf0,
                    o_hbm.at[pl.ds(base + _HEAD, _CHUNK - _HEAD)], outsems[0])
            return pltpu.make_async_copy(
                bufs[c % _NBUF],
                o_hbm.at[pl.ds(base + c * _CHUNK, _CHUNK)], outsems[c % _NBUF])

        for c in range(_D):
            in_cp(c).start()
        for c in range(_NCHUNK):
            in_cp(c).wait()
            out_cp(c).start()
            nxt = c + _D
            if nxt < _NCHUNK:
                if nxt >= _NBUF:
                    out_cp(nxt - _NBUF).wait()
                in_cp(nxt).start()
        for c in range(_NCHUNK - _NBUF, _NCHUNK):
            out_cp(c).wait()

    pl.run_scoped(
        inner,
        *([pltpu.VMEM_SHARED((_CHUNK,), jnp.float32)] * _NBUF),
        pltpu.VMEM_SHARED((_CHUNK - _HEAD,), jnp.float32),
        *([pltpu.SemaphoreType.DMA] * (2 * _NBUF)),
    )


def _vector_body(x_hbm, o_hbm):
    def inner(buf16):
        core = lax.axis_index("c")
        sid = lax.axis_index("s")
        base = core * _SHARD

        @pl.when(sid == 0)
        def _head():
            pltpu.sync_copy(x_hbm.at[pl.ds(base, _HEAD)], buf16)

            @pl.when(core == 0)
            def _patch():
                i = lax.iota(jnp.int32, 16)
                v = buf16[pl.ds(0, 16)]
                buf16[pl.ds(0, 16)] = jnp.where(i == 0, 10.0,
                                      jnp.where(i == 1, 30.0,
                                      jnp.where(i == 2, 20.0,
                                      jnp.where(i == 3, 40.0, v))))

            pltpu.sync_copy(buf16, o_hbm.at[pl.ds(base, _HEAD)])

    pl.run_scoped(inner, pltpu.VMEM((_HEAD,), jnp.float32))


def kernel(data):
    s_mesh = plsc.ScalarSubcoreMesh(axis_name="c")
    v_mesh = plsc.VectorSubcoreMesh(core_axis_name="c", subcore_axis_name="s")
    f = pl.kernel(
        body=[_scalar_body, _vector_body],
        mesh=[s_mesh, v_mesh],
        out_type=jax.ShapeDtypeStruct((_N,), jnp.float32),
    )
    return f(data)
